# baseline (speedup 1.0000x reference)
import functools

import jax
import jax.numpy as jnp
from jax import lax
from jax.experimental import pallas as pl
from jax.experimental.pallas import tpu as pltpu
from jax.experimental.pallas import tpu_sc as plsc

N_N = 10000
D_F = 256
N_E = 160000
HALF = 128
N_PAD = 10240
E_PAD = 163840
CH = 128
CPT = E_PAD // CH // 16
RPT = N_PAD // 16
RCH = RPT // CH

_SC_MESH = plsc.VectorSubcoreMesh(core_axis_name="c", subcore_axis_name="s")


def _fill(ref, n, value):
    val = jnp.full((16,), value, jnp.float32)

    def body(i, _):
        ref[pl.ds(i * 16, 16)] = val
        return 0

    lax.fori_loop(0, n // 16, body, 0)


CPT_DEG = CPT // 2


def _deg_body(dst2d, deg0_out, deg1_out, didx, ones, dbuf, shared_deg):
    c = lax.axis_index("c")
    s = lax.axis_index("s")

    def run(deg_out):
        _fill(ones, CH, 1.0)
        _fill(dbuf, RPT, 0.0)
        pltpu.sync_copy(dbuf, shared_deg.at[pl.ds(s * RPT, RPT)])
        base = c * (E_PAD // CH // 2) + s * CPT_DEG
        pltpu.sync_copy(dst2d.at[pl.ds(base, CPT_DEG), :], didx)
        plsc.subcore_barrier()

        def body(j, _):
            pltpu.sync_copy(ones, shared_deg.at[didx.at[j]], add=True)
            return 0

        lax.fori_loop(0, CPT_DEG, body, 0)
        plsc.subcore_barrier()
        pltpu.sync_copy(shared_deg.at[pl.ds(s * RPT, RPT)], dbuf)
        pltpu.sync_copy(dbuf, deg_out.at[pl.ds(s * RPT, RPT)])

    @pl.when(c == 0)
    def _():
        run(deg0_out)

    @pl.when(c == 1)
    def _():
        run(deg1_out)


_deg_call = functools.partial(
    pl.kernel,
    out_type=(
        jax.ShapeDtypeStruct((N_PAD,), jnp.float32),
        jax.ShapeDtypeStruct((N_PAD,), jnp.float32),
    ),
    mesh=_SC_MESH,
    scratch_types=[
        pltpu.VMEM((CPT_DEG, CH), jnp.int32),
        pltpu.VMEM((CH,), jnp.float32),
        pltpu.VMEM((RPT,), jnp.float32),
        pltpu.VMEM_SHARED((N_PAD,), jnp.float32),
    ],
)(_deg_body)


PHC = CPT // 2


def _agg_body(src2d, dst2d, ga, gb, agga, aggb, sidx, didx, rows0, rows1,
              shared_agg, gsem0, gsem1):
    c = lax.axis_index("c")
    s = lax.axis_index("s")

    def run(g_hbm, agg_hbm):
        n_init = jnp.where(s == 15, 5, 8)

        def init(k, _):
            start = s * RPT + k * 80
            pltpu.sync_copy(g_hbm.at[pl.ds(start, 80), :],
                            rows0.at[pl.ds(0, 80), :])
            pltpu.sync_copy(rows0.at[pl.ds(0, 80), :],
                            shared_agg.at[pl.ds(start, 80), :])
            return 0

        lax.fori_loop(0, n_init, init, 0)
        plsc.subcore_barrier()

        def phase(p, _):
            base = s * CPT + p * PHC
            pltpu.sync_copy(src2d.at[pl.ds(base, PHC), :], sidx)
            pltpu.sync_copy(dst2d.at[pl.ds(base, PHC), :], didx)
            pltpu.async_copy(g_hbm.at[sidx.at[0]], rows0, gsem0)
            pltpu.async_copy(g_hbm.at[sidx.at[1]], rows1, gsem1)

            def body(i, _):
                j0 = 2 * i
                j1 = j0 + 1
                pltpu.make_async_copy(
                    g_hbm.at[sidx.at[j0]], rows0, gsem0).wait()
                pltpu.sync_copy(rows0, shared_agg.at[didx.at[j0]], add=True)

                @pl.when(j0 + 2 < PHC)
                def _():
                    pltpu.async_copy(g_hbm.at[sidx.at[j0 + 2]], rows0, gsem0)

                pltpu.make_async_copy(
                    g_hbm.at[sidx.at[j1]], rows1, gsem1).wait()
                pltpu.sync_copy(rows1, shared_agg.at[didx.at[j1]], add=True)

                @pl.when(j1 + 2 < PHC)
                def _():
                    pltpu.async_copy(g_hbm.at[sidx.at[j1 + 2]], rows1, gsem1)

                return 0

            lax.fori_loop(0, PHC // 2, body, 0)
            return 0

        lax.fori_loop(0, CPT // PHC, phase, 0)
        plsc.subcore_barrier()

        def drain(k, _):
            start = s * RPT + k * CH
            pltpu.sync_copy(shared_agg.at[pl.ds(start, CH), :], rows0)
            pltpu.sync_copy(rows0, agg_hbm.at[pl.ds(start, CH), :])
            return 0

        lax.fori_loop(0, RCH, drain, 0)

    @pl.when(c == 0)
    def _():
        run(ga, agga)

    @pl.when(c == 1)
    def _():
        run(gb, aggb)


_agg_call = functools.partial(
    pl.kernel,
    out_type=(
        jax.ShapeDtypeStruct((N_PAD, HALF), jnp.float32),
        jax.ShapeDtypeStruct((N_PAD, HALF), jnp.float32),
    ),
    mesh=_SC_MESH,
    scratch_types=[
        pltpu.VMEM((PHC, CH), jnp.int32),
        pltpu.VMEM((PHC, CH), jnp.int32),
        pltpu.VMEM((CH, HALF), jnp.float32),
        pltpu.VMEM((CH, HALF), jnp.float32),
        pltpu.VMEM_SHARED((N_PAD, HALF), jnp.float32),
        pltpu.SemaphoreType.DMA,
        pltpu.SemaphoreType.DMA,
    ],
)(_agg_body)


_ROWS = 400
_GRID = N_N // _ROWS


def _mm_body(deg0_ref, deg1_ref, x_ref, w_ref, ga_ref, gb_ref, dinv_ref):
    dinv = lax.rsqrt(deg0_ref[...] + deg1_ref[...] + 1.0)
    h = jnp.dot(x_ref[...], w_ref[...], preferred_element_type=jnp.float32)
    g = h * dinv
    ga_ref[...] = g[:, :HALF]
    gb_ref[...] = g[:, HALF:]
    dinv_ref[...] = dinv


_mm_call = pl.pallas_call(
    _mm_body,
    grid=(_GRID,),
    in_specs=[
        pl.BlockSpec((_ROWS, 1), lambda i: (i, 0)),
        pl.BlockSpec((_ROWS, 1), lambda i: (i, 0)),
        pl.BlockSpec((_ROWS, D_F), lambda i: (i, 0)),
        pl.BlockSpec((D_F, D_F), lambda i: (0, 0)),
    ],
    out_specs=[
        pl.BlockSpec((_ROWS, HALF), lambda i: (i, 0)),
        pl.BlockSpec((_ROWS, HALF), lambda i: (i, 0)),
        pl.BlockSpec((_ROWS, 1), lambda i: (i, 0)),
    ],
    out_shape=[
        jax.ShapeDtypeStruct((N_N, HALF), jnp.float32),
        jax.ShapeDtypeStruct((N_N, HALF), jnp.float32),
        jax.ShapeDtypeStruct((N_N, 1), jnp.float32),
    ],
)


def _post_body(x_ref, aa_ref, ab_ref, dinv_ref, bc_ref, lg_ref, lb_ref,
               wnn_ref, bnn_ref, out_ref):
    agg = jnp.concatenate([aa_ref[...], ab_ref[...]], axis=1)
    pre = agg * dinv_ref[...] + bc_ref[...]
    mu = jnp.mean(pre, axis=1, keepdims=True)
    d = pre - mu
    var = jnp.mean(d * d, axis=1, keepdims=True)
    ln = d * lax.rsqrt(var + 1e-5) * lg_ref[...] + lb_ref[...]
    r = jnp.maximum(x_ref[...] + ln, 0.0)
    out_ref[...] = (
        jnp.dot(r, wnn_ref[...], preferred_element_type=jnp.float32)
        + bnn_ref[...]
    )


_post_call = pl.pallas_call(
    _post_body,
    grid=(_GRID,),
    in_specs=[
        pl.BlockSpec((_ROWS, D_F), lambda i: (i, 0)),
        pl.BlockSpec((_ROWS, HALF), lambda i: (i, 0)),
        pl.BlockSpec((_ROWS, HALF), lambda i: (i, 0)),
        pl.BlockSpec((_ROWS, 1), lambda i: (i, 0)),
        pl.BlockSpec((1, D_F), lambda i: (0, 0)),
        pl.BlockSpec((1, D_F), lambda i: (0, 0)),
        pl.BlockSpec((1, D_F), lambda i: (0, 0)),
        pl.BlockSpec((D_F, D_F), lambda i: (0, 0)),
        pl.BlockSpec((1, D_F), lambda i: (0, 0)),
    ],
    out_specs=pl.BlockSpec((_ROWS, D_F), lambda i: (i, 0)),
    out_shape=jax.ShapeDtypeStruct((N_N, D_F), jnp.float32),
)


def kernel(x, edge_index, W_conv, b_conv, ln_gamma, ln_beta, W_nn, b_nn):
    ei = edge_index.astype(jnp.int32)
    src2d = jnp.concatenate(
        [ei[0], jnp.zeros((E_PAD - N_E,), jnp.int32)]).reshape(E_PAD // CH, CH)
    dst2d = jnp.concatenate(
        [ei[1], jnp.full((E_PAD - N_E,), N_PAD - 1, jnp.int32)]
    ).reshape(E_PAD // CH, CH)

    deg0, deg1 = _deg_call(dst2d)
    ga, gb, dinv = _mm_call(
        deg0.reshape(N_PAD, 1), deg1.reshape(N_PAD, 1), x, W_conv)
    agga, aggb = _agg_call(src2d, dst2d, ga, gb)
    return _post_call(
        x, agga, aggb, dinv,
        b_conv.reshape(1, D_F), ln_gamma.reshape(1, D_F),
        ln_beta.reshape(1, D_F), W_nn, b_nn.reshape(1, D_F),
    )


# speedup vs baseline: 1.0679x; 1.0679x over previous
import functools

import jax
import jax.numpy as jnp
from jax import lax
from jax.experimental import pallas as pl
from jax.experimental.pallas import tpu as pltpu
from jax.experimental.pallas import tpu_sc as plsc

N_N = 10000
D_F = 256
N_E = 160000
HALF = 128
N_PAD = 10240
E_PAD = 163840
CH = 128
CPT = E_PAD // CH // 16
RPT = N_PAD // 16
RCH = RPT // CH

_SC_MESH = plsc.VectorSubcoreMesh(core_axis_name="c", subcore_axis_name="s")


def _fill(ref, n, value):
    val = jnp.full((16,), value, jnp.float32)

    def body(i, _):
        ref[pl.ds(i * 16, 16)] = val
        return 0

    lax.fori_loop(0, n // 16, body, 0)


CPT_DEG = CPT // 2


def _deg_body(dst2d, deg0_out, deg1_out, didx, ones, dbuf, shared_deg):
    c = lax.axis_index("c")
    s = lax.axis_index("s")

    def run(deg_out):
        _fill(ones, CH, 1.0)
        _fill(dbuf, RPT, 0.0)
        pltpu.sync_copy(dbuf, shared_deg.at[pl.ds(s * RPT, RPT)])
        base = c * (E_PAD // CH // 2) + s * CPT_DEG
        pltpu.sync_copy(dst2d.at[pl.ds(base, CPT_DEG), :], didx)
        plsc.subcore_barrier()

        def body(j, _):
            pltpu.sync_copy(ones, shared_deg.at[didx.at[j]], add=True)
            return 0

        lax.fori_loop(0, CPT_DEG, body, 0)
        plsc.subcore_barrier()
        pltpu.sync_copy(shared_deg.at[pl.ds(s * RPT, RPT)], dbuf)
        pltpu.sync_copy(dbuf, deg_out.at[pl.ds(s * RPT, RPT)])

    @pl.when(c == 0)
    def _():
        run(deg0_out)

    @pl.when(c == 1)
    def _():
        run(deg1_out)


_deg_call = functools.partial(
    pl.kernel,
    out_type=(
        jax.ShapeDtypeStruct((N_PAD,), jnp.float32),
        jax.ShapeDtypeStruct((N_PAD,), jnp.float32),
    ),
    mesh=_SC_MESH,
    scratch_types=[
        pltpu.VMEM((CPT_DEG, CH), jnp.int32),
        pltpu.VMEM((CH,), jnp.float32),
        pltpu.VMEM((RPT,), jnp.float32),
        pltpu.VMEM_SHARED((N_PAD,), jnp.float32),
    ],
)(_deg_body)


PHC = CPT // 2


def _agg_body(src2d, dst2d, ga, gb, agga, aggb, sidx, didx, rows0, rows1,
              shared_agg, gsem0, gsem1):
    c = lax.axis_index("c")
    s = lax.axis_index("s")

    def run(g_hbm, agg_hbm):
        def init(k, _):
            start = s * RPT + k * CH
            pltpu.sync_copy(g_hbm.at[pl.ds(start, CH), :], rows0)
            pltpu.sync_copy(rows0, shared_agg.at[pl.ds(start, CH), :])
            return 0

        lax.fori_loop(0, RCH, init, 0)
        plsc.subcore_barrier()

        def phase(p, _):
            base = s * CPT + p * PHC
            pltpu.sync_copy(src2d.at[pl.ds(base, PHC), :], sidx)
            pltpu.sync_copy(dst2d.at[pl.ds(base, PHC), :], didx)
            pltpu.async_copy(g_hbm.at[sidx.at[0]], rows0, gsem0)
            pltpu.async_copy(g_hbm.at[sidx.at[1]], rows1, gsem1)

            def body(i, _):
                j0 = 2 * i
                j1 = j0 + 1
                pltpu.make_async_copy(
                    g_hbm.at[sidx.at[j0]], rows0, gsem0).wait()
                pltpu.sync_copy(rows0, shared_agg.at[didx.at[j0]], add=True)

                @pl.when(j0 + 2 < PHC)
                def _():
                    pltpu.async_copy(g_hbm.at[sidx.at[j0 + 2]], rows0, gsem0)

                pltpu.make_async_copy(
                    g_hbm.at[sidx.at[j1]], rows1, gsem1).wait()
                pltpu.sync_copy(rows1, shared_agg.at[didx.at[j1]], add=True)

                @pl.when(j1 + 2 < PHC)
                def _():
                    pltpu.async_copy(g_hbm.at[sidx.at[j1 + 2]], rows1, gsem1)

                return 0

            lax.fori_loop(0, PHC // 2, body, 0)
            return 0

        lax.fori_loop(0, CPT // PHC, phase, 0)
        plsc.subcore_barrier()

        def drain(k, _):
            start = s * RPT + k * CH
            pltpu.sync_copy(shared_agg.at[pl.ds(start, CH), :], rows0)
            pltpu.sync_copy(rows0, agg_hbm.at[pl.ds(start, CH), :])
            return 0

        lax.fori_loop(0, RCH, drain, 0)

    @pl.when(c == 0)
    def _():
        run(ga, agga)

    @pl.when(c == 1)
    def _():
        run(gb, aggb)


_agg_call = functools.partial(
    pl.kernel,
    out_type=(
        jax.ShapeDtypeStruct((N_PAD, HALF), jnp.float32),
        jax.ShapeDtypeStruct((N_PAD, HALF), jnp.float32),
    ),
    mesh=_SC_MESH,
    scratch_types=[
        pltpu.VMEM((PHC, CH), jnp.int32),
        pltpu.VMEM((PHC, CH), jnp.int32),
        pltpu.VMEM((CH, HALF), jnp.float32),
        pltpu.VMEM((CH, HALF), jnp.float32),
        pltpu.VMEM_SHARED((N_PAD, HALF), jnp.float32),
        pltpu.SemaphoreType.DMA,
        pltpu.SemaphoreType.DMA,
    ],
)(_agg_body)


_ROWS = 512
_GRID = N_PAD // _ROWS


def _mm_body(deg0_ref, deg1_ref, x_ref, w_ref, ga_ref, gb_ref, dinv_ref):
    dinv = lax.rsqrt(deg0_ref[...] + deg1_ref[...] + 1.0)
    h = jnp.dot(x_ref[...], w_ref[...], preferred_element_type=jnp.float32)
    g = h * dinv
    ga_ref[...] = g[:, :HALF]
    gb_ref[...] = g[:, HALF:]
    dinv_ref[...] = dinv


_mm_call = pl.pallas_call(
    _mm_body,
    grid=(_GRID,),
    in_specs=[
        pl.BlockSpec((_ROWS, 1), lambda i: (i, 0)),
        pl.BlockSpec((_ROWS, 1), lambda i: (i, 0)),
        pl.BlockSpec((_ROWS, D_F), lambda i: (i, 0)),
        pl.BlockSpec((D_F, D_F), lambda i: (0, 0)),
    ],
    out_specs=[
        pl.BlockSpec((_ROWS, HALF), lambda i: (i, 0)),
        pl.BlockSpec((_ROWS, HALF), lambda i: (i, 0)),
        pl.BlockSpec((_ROWS, 1), lambda i: (i, 0)),
    ],
    out_shape=[
        jax.ShapeDtypeStruct((N_PAD, HALF), jnp.float32),
        jax.ShapeDtypeStruct((N_PAD, HALF), jnp.float32),
        jax.ShapeDtypeStruct((N_PAD, 1), jnp.float32),
    ],
)


def _post_body(x_ref, aa_ref, ab_ref, dinv_ref, bc_ref, lg_ref, lb_ref,
               wnn_ref, bnn_ref, out_ref):
    agg = jnp.concatenate([aa_ref[...], ab_ref[...]], axis=1)
    pre = agg * dinv_ref[...] + bc_ref[...]
    mu = jnp.mean(pre, axis=1, keepdims=True)
    d = pre - mu
    var = jnp.mean(d * d, axis=1, keepdims=True)
    ln = d * lax.rsqrt(var + 1e-5) * lg_ref[...] + lb_ref[...]
    r = jnp.maximum(x_ref[...] + ln, 0.0)
    out_ref[...] = (
        jnp.dot(r, wnn_ref[...], preferred_element_type=jnp.float32)
        + bnn_ref[...]
    )


_post_call = pl.pallas_call(
    _post_body,
    grid=(_GRID,),
    in_specs=[
        pl.BlockSpec((_ROWS, D_F), lambda i: (i, 0)),
        pl.BlockSpec((_ROWS, HALF), lambda i: (i, 0)),
        pl.BlockSpec((_ROWS, HALF), lambda i: (i, 0)),
        pl.BlockSpec((_ROWS, 1), lambda i: (i, 0)),
        pl.BlockSpec((1, D_F), lambda i: (0, 0)),
        pl.BlockSpec((1, D_F), lambda i: (0, 0)),
        pl.BlockSpec((1, D_F), lambda i: (0, 0)),
        pl.BlockSpec((D_F, D_F), lambda i: (0, 0)),
        pl.BlockSpec((1, D_F), lambda i: (0, 0)),
    ],
    out_specs=pl.BlockSpec((_ROWS, D_F), lambda i: (i, 0)),
    out_shape=jax.ShapeDtypeStruct((N_PAD, D_F), jnp.float32),
)


def kernel(x, edge_index, W_conv, b_conv, ln_gamma, ln_beta, W_nn, b_nn):
    ei = edge_index.astype(jnp.int32)
    src2d = jnp.concatenate(
        [ei[0], jnp.zeros((E_PAD - N_E,), jnp.int32)]).reshape(E_PAD // CH, CH)
    dst2d = jnp.concatenate(
        [ei[1], jnp.full((E_PAD - N_E,), N_PAD - 1, jnp.int32)]
    ).reshape(E_PAD // CH, CH)

    x_pad = jnp.pad(x, ((0, N_PAD - N_N), (0, 0)))
    deg0, deg1 = _deg_call(dst2d)
    ga, gb, dinv = _mm_call(
        deg0.reshape(N_PAD, 1), deg1.reshape(N_PAD, 1), x_pad, W_conv)
    agga, aggb = _agg_call(src2d, dst2d, ga, gb)
    out = _post_call(
        x_pad, agga, aggb, dinv,
        b_conv.reshape(1, D_F), ln_gamma.reshape(1, D_F),
        ln_beta.reshape(1, D_F), W_nn, b_nn.reshape(1, D_F),
    )
    return out[:N_N]


# speedup vs baseline: 1.1070x; 1.0367x over previous
import functools

import jax
import jax.numpy as jnp
from jax import lax
from jax.experimental import pallas as pl
from jax.experimental.pallas import tpu as pltpu
from jax.experimental.pallas import tpu_sc as plsc

N_N = 10000
D_F = 256
N_E = 160000
HALF = 128
N_PAD = 10240
E_PAD = 163840
CH = 128
CPT = E_PAD // CH // 16
RPT = N_PAD // 16
RCH = RPT // CH

_SC_MESH = plsc.VectorSubcoreMesh(core_axis_name="c", subcore_axis_name="s")


def _fill(ref, n, value):
    val = jnp.full((16,), value, jnp.float32)

    def body(i, _):
        ref[pl.ds(i * 16, 16)] = val
        return 0

    lax.fori_loop(0, n // 16, body, 0)


CPT_DEG = CPT // 2


def _deg_body(dst2d, deg0_out, deg1_out, didx, ones, dbuf, shared_deg):
    c = lax.axis_index("c")
    s = lax.axis_index("s")

    def run(deg_out):
        _fill(ones, CH, 1.0)
        _fill(dbuf, RPT, 0.0)
        pltpu.sync_copy(dbuf, shared_deg.at[pl.ds(s * RPT, RPT)])
        base = c * (E_PAD // CH // 2) + s * CPT_DEG
        pltpu.sync_copy(dst2d.at[pl.ds(base, CPT_DEG), :], didx)
        plsc.subcore_barrier()

        def body(j, _):
            pltpu.sync_copy(ones, shared_deg.at[didx.at[j]], add=True)
            return 0

        lax.fori_loop(0, CPT_DEG, body, 0)
        plsc.subcore_barrier()
        pltpu.sync_copy(shared_deg.at[pl.ds(s * RPT, RPT)], dbuf)
        pltpu.sync_copy(dbuf, deg_out.at[pl.ds(s * RPT, RPT)])

    @pl.when(c == 0)
    def _():
        run(deg0_out)

    @pl.when(c == 1)
    def _():
        run(deg1_out)


_deg_call = functools.partial(
    pl.kernel,
    out_type=(
        jax.ShapeDtypeStruct((N_PAD,), jnp.float32),
        jax.ShapeDtypeStruct((N_PAD,), jnp.float32),
    ),
    mesh=_SC_MESH,
    scratch_types=[
        pltpu.VMEM((CPT_DEG, CH), jnp.int32),
        pltpu.VMEM((CH,), jnp.float32),
        pltpu.VMEM((RPT,), jnp.float32),
        pltpu.VMEM_SHARED((N_PAD,), jnp.float32),
    ],
)(_deg_body)


PHC = CPT // 2


def _agg_body(src2d, dst2d, ga, gb, agga, aggb, sidx, didx, rows0, rows1,
              shared_agg, gsem0, gsem1):
    c = lax.axis_index("c")
    s = lax.axis_index("s")

    def run(g_hbm, agg_hbm):
        def init(k, _):
            start = s * RPT + k * CH
            pltpu.sync_copy(g_hbm.at[pl.ds(start, CH), :], rows0)
            pltpu.sync_copy(rows0, shared_agg.at[pl.ds(start, CH), :])
            return 0

        lax.fori_loop(0, RCH, init, 0)
        plsc.subcore_barrier()

        def phase(p, _):
            base = s * CPT + p * PHC
            pltpu.sync_copy(src2d.at[pl.ds(base, PHC), :], sidx)
            pltpu.sync_copy(dst2d.at[pl.ds(base, PHC), :], didx)
            pltpu.async_copy(g_hbm.at[sidx.at[0]], rows0, gsem0)
            pltpu.async_copy(g_hbm.at[sidx.at[1]], rows1, gsem1)

            def body(i, _):
                j0 = 2 * i
                j1 = j0 + 1
                pltpu.make_async_copy(
                    g_hbm.at[sidx.at[j0]], rows0, gsem0).wait()
                pltpu.sync_copy(rows0, shared_agg.at[didx.at[j0]], add=True)

                @pl.when(j0 + 2 < PHC)
                def _():
                    pltpu.async_copy(g_hbm.at[sidx.at[j0 + 2]], rows0, gsem0)

                pltpu.make_async_copy(
                    g_hbm.at[sidx.at[j1]], rows1, gsem1).wait()
                pltpu.sync_copy(rows1, shared_agg.at[didx.at[j1]], add=True)

                @pl.when(j1 + 2 < PHC)
                def _():
                    pltpu.async_copy(g_hbm.at[sidx.at[j1 + 2]], rows1, gsem1)

                return 0

            lax.fori_loop(0, PHC // 2, body, 0)
            return 0

        lax.fori_loop(0, CPT // PHC, phase, 0)
        plsc.subcore_barrier()

        def drain(k, _):
            start = s * RPT + k * CH
            pltpu.sync_copy(shared_agg.at[pl.ds(start, CH), :], rows0)
            pltpu.sync_copy(rows0, agg_hbm.at[pl.ds(start, CH), :])
            return 0

        lax.fori_loop(0, RCH, drain, 0)

    @pl.when(c == 0)
    def _():
        run(ga, agga)

    @pl.when(c == 1)
    def _():
        run(gb, aggb)


_agg_call = functools.partial(
    pl.kernel,
    out_type=(
        jax.ShapeDtypeStruct((N_PAD, HALF), jnp.float32),
        jax.ShapeDtypeStruct((N_PAD, HALF), jnp.float32),
    ),
    mesh=_SC_MESH,
    scratch_types=[
        pltpu.VMEM((PHC, CH), jnp.int32),
        pltpu.VMEM((PHC, CH), jnp.int32),
        pltpu.VMEM((CH, HALF), jnp.float32),
        pltpu.VMEM((CH, HALF), jnp.float32),
        pltpu.VMEM_SHARED((N_PAD, HALF), jnp.float32),
        pltpu.SemaphoreType.DMA,
        pltpu.SemaphoreType.DMA,
    ],
)(_agg_body)


_ROWS = 1024
_GRID = N_PAD // _ROWS


def _mm_body(deg0_ref, deg1_ref, x_ref, w_ref, ga_ref, gb_ref, dinv_ref):
    dinv = lax.rsqrt(deg0_ref[...] + deg1_ref[...] + 1.0)
    h = jnp.dot(x_ref[...], w_ref[...], preferred_element_type=jnp.float32)
    g = h * dinv
    ga_ref[...] = g[:, :HALF]
    gb_ref[...] = g[:, HALF:]
    dinv_ref[...] = dinv


_mm_call = pl.pallas_call(
    _mm_body,
    grid=(_GRID,),
    in_specs=[
        pl.BlockSpec((_ROWS, 1), lambda i: (i, 0)),
        pl.BlockSpec((_ROWS, 1), lambda i: (i, 0)),
        pl.BlockSpec((_ROWS, D_F), lambda i: (i, 0)),
        pl.BlockSpec((D_F, D_F), lambda i: (0, 0)),
    ],
    out_specs=[
        pl.BlockSpec((_ROWS, HALF), lambda i: (i, 0)),
        pl.BlockSpec((_ROWS, HALF), lambda i: (i, 0)),
        pl.BlockSpec((_ROWS, 1), lambda i: (i, 0)),
    ],
    out_shape=[
        jax.ShapeDtypeStruct((N_PAD, HALF), jnp.float32),
        jax.ShapeDtypeStruct((N_PAD, HALF), jnp.float32),
        jax.ShapeDtypeStruct((N_PAD, 1), jnp.float32),
    ],
)


def _post_body(x_ref, aa_ref, ab_ref, dinv_ref, bc_ref, lg_ref, lb_ref,
               wnn_ref, bnn_ref, out_ref):
    agg = jnp.concatenate([aa_ref[...], ab_ref[...]], axis=1)
    pre = agg * dinv_ref[...] + bc_ref[...]
    mu = jnp.mean(pre, axis=1, keepdims=True)
    d = pre - mu
    var = jnp.mean(d * d, axis=1, keepdims=True)
    ln = d * lax.rsqrt(var + 1e-5) * lg_ref[...] + lb_ref[...]
    r = jnp.maximum(x_ref[...] + ln, 0.0)
    out_ref[...] = (
        jnp.dot(r, wnn_ref[...], preferred_element_type=jnp.float32)
        + bnn_ref[...]
    )


_post_call = pl.pallas_call(
    _post_body,
    grid=(_GRID,),
    in_specs=[
        pl.BlockSpec((_ROWS, D_F), lambda i: (i, 0)),
        pl.BlockSpec((_ROWS, HALF), lambda i: (i, 0)),
        pl.BlockSpec((_ROWS, HALF), lambda i: (i, 0)),
        pl.BlockSpec((_ROWS, 1), lambda i: (i, 0)),
        pl.BlockSpec((1, D_F), lambda i: (0, 0)),
        pl.BlockSpec((1, D_F), lambda i: (0, 0)),
        pl.BlockSpec((1, D_F), lambda i: (0, 0)),
        pl.BlockSpec((D_F, D_F), lambda i: (0, 0)),
        pl.BlockSpec((1, D_F), lambda i: (0, 0)),
    ],
    out_specs=pl.BlockSpec((_ROWS, D_F), lambda i: (i, 0)),
    out_shape=jax.ShapeDtypeStruct((N_PAD, D_F), jnp.float32),
)


def kernel(x, edge_index, W_conv, b_conv, ln_gamma, ln_beta, W_nn, b_nn):
    ei = edge_index.astype(jnp.int32)
    src2d = jnp.concatenate(
        [ei[0], jnp.zeros((E_PAD - N_E,), jnp.int32)]).reshape(E_PAD // CH, CH)
    dst2d = jnp.concatenate(
        [ei[1], jnp.full((E_PAD - N_E,), N_PAD - 1, jnp.int32)]
    ).reshape(E_PAD // CH, CH)

    x_pad = jnp.pad(x, ((0, N_PAD - N_N), (0, 0)))
    deg0, deg1 = _deg_call(dst2d)
    ga, gb, dinv = _mm_call(
        deg0.reshape(N_PAD, 1), deg1.reshape(N_PAD, 1), x_pad, W_conv)
    agga, aggb = _agg_call(src2d, dst2d, ga, gb)
    out = _post_call(
        x_pad, agga, aggb, dinv,
        b_conv.reshape(1, D_F), ln_gamma.reshape(1, D_F),
        ln_beta.reshape(1, D_F), W_nn, b_nn.reshape(1, D_F),
    )
    return out[:N_N]


# speedup vs baseline: 1.1197x; 1.0115x over previous
import functools

import jax
import jax.numpy as jnp
from jax import lax
from jax.experimental import pallas as pl
from jax.experimental.pallas import tpu as pltpu
from jax.experimental.pallas import tpu_sc as plsc

N_N = 10000
D_F = 256
N_E = 160000
HALF = 128
N_PAD = 10240
E_PAD = 163840
CH = 128
CPT = E_PAD // CH // 16
RPT = N_PAD // 16
RCH = RPT // CH

_SC_MESH = plsc.VectorSubcoreMesh(core_axis_name="c", subcore_axis_name="s")


def _fill(ref, n, value):
    val = jnp.full((16,), value, jnp.float32)

    def body(i, _):
        ref[pl.ds(i * 16, 16)] = val
        return 0

    lax.fori_loop(0, n // 16, body, 0)


CPT_DEG = CPT // 2


def _deg_body(dst2d, deg0_out, deg1_out, didx, ones, dbuf, shared_deg):
    c = lax.axis_index("c")
    s = lax.axis_index("s")

    def run(deg_out):
        _fill(ones, CH, 1.0)
        _fill(dbuf, RPT, 0.0)
        pltpu.sync_copy(dbuf, shared_deg.at[pl.ds(s * RPT, RPT)])
        base = c * (E_PAD // CH // 2) + s * CPT_DEG
        pltpu.sync_copy(dst2d.at[pl.ds(base, CPT_DEG), :], didx)
        plsc.subcore_barrier()

        def body(j, _):
            pltpu.sync_copy(ones, shared_deg.at[didx.at[j]], add=True)
            return 0

        lax.fori_loop(0, CPT_DEG, body, 0)
        plsc.subcore_barrier()
        pltpu.sync_copy(shared_deg.at[pl.ds(s * RPT, RPT)], dbuf)
        pltpu.sync_copy(dbuf, deg_out.at[pl.ds(s * RPT, RPT)])

    @pl.when(c == 0)
    def _():
        run(deg0_out)

    @pl.when(c == 1)
    def _():
        run(deg1_out)


_deg_call = functools.partial(
    pl.kernel,
    out_type=(
        jax.ShapeDtypeStruct((N_PAD,), jnp.float32),
        jax.ShapeDtypeStruct((N_PAD,), jnp.float32),
    ),
    mesh=_SC_MESH,
    scratch_types=[
        pltpu.VMEM((CPT_DEG, CH), jnp.int32),
        pltpu.VMEM((CH,), jnp.float32),
        pltpu.VMEM((RPT,), jnp.float32),
        pltpu.VMEM_SHARED((N_PAD,), jnp.float32),
    ],
)(_deg_body)


PHC = CPT // 2


def _agg_body(src2d, dst2d, ga, gb, agga, aggb, sidx, didx, rows0, rows1,
              shared_agg, gsem0, gsem1):
    c = lax.axis_index("c")
    s = lax.axis_index("s")

    def run(g_hbm, agg_hbm):
        def init(k, _):
            start = s * RPT + k * CH
            pltpu.sync_copy(g_hbm.at[pl.ds(start, CH), :], rows0)
            pltpu.sync_copy(rows0, shared_agg.at[pl.ds(start, CH), :])
            return 0

        lax.fori_loop(0, RCH, init, 0)
        plsc.subcore_barrier()

        def phase(p, _):
            base = s * CPT + p * PHC
            pltpu.sync_copy(src2d.at[pl.ds(base, PHC), :], sidx)
            pltpu.sync_copy(dst2d.at[pl.ds(base, PHC), :], didx)
            pltpu.async_copy(g_hbm.at[sidx.at[0]], rows0, gsem0)
            pltpu.async_copy(g_hbm.at[sidx.at[1]], rows1, gsem1)

            def body(i, _):
                j0 = 2 * i
                j1 = j0 + 1
                pltpu.make_async_copy(
                    g_hbm.at[sidx.at[j0]], rows0, gsem0).wait()
                pltpu.sync_copy(rows0, shared_agg.at[didx.at[j0]], add=True)

                @pl.when(j0 + 2 < PHC)
                def _():
                    pltpu.async_copy(g_hbm.at[sidx.at[j0 + 2]], rows0, gsem0)

                pltpu.make_async_copy(
                    g_hbm.at[sidx.at[j1]], rows1, gsem1).wait()
                pltpu.sync_copy(rows1, shared_agg.at[didx.at[j1]], add=True)

                @pl.when(j1 + 2 < PHC)
                def _():
                    pltpu.async_copy(g_hbm.at[sidx.at[j1 + 2]], rows1, gsem1)

                return 0

            lax.fori_loop(0, PHC // 2, body, 0)
            return 0

        lax.fori_loop(0, CPT // PHC, phase, 0)
        plsc.subcore_barrier()

        def drain(k, _):
            start = s * RPT + k * CH
            pltpu.sync_copy(shared_agg.at[pl.ds(start, CH), :], rows0)
            pltpu.sync_copy(rows0, agg_hbm.at[pl.ds(start, CH), :])
            return 0

        lax.fori_loop(0, RCH, drain, 0)

    @pl.when(c == 0)
    def _():
        run(ga, agga)

    @pl.when(c == 1)
    def _():
        run(gb, aggb)


_agg_call = functools.partial(
    pl.kernel,
    out_type=(
        jax.ShapeDtypeStruct((N_PAD, HALF), jnp.float32),
        jax.ShapeDtypeStruct((N_PAD, HALF), jnp.float32),
    ),
    mesh=_SC_MESH,
    scratch_types=[
        pltpu.VMEM((PHC, CH), jnp.int32),
        pltpu.VMEM((PHC, CH), jnp.int32),
        pltpu.VMEM((CH, HALF), jnp.float32),
        pltpu.VMEM((CH, HALF), jnp.float32),
        pltpu.VMEM_SHARED((N_PAD, HALF), jnp.float32),
        pltpu.SemaphoreType.DMA,
        pltpu.SemaphoreType.DMA,
    ],
)(_agg_body)


_ROWS = 2048
_GRID = N_PAD // _ROWS


def _mm_body(deg0_ref, deg1_ref, x_ref, w_ref, ga_ref, gb_ref, dinv_ref):
    dinv = lax.rsqrt(deg0_ref[...] + deg1_ref[...] + 1.0)
    h = jnp.dot(x_ref[...], w_ref[...], preferred_element_type=jnp.float32)
    g = h * dinv
    ga_ref[...] = g[:, :HALF]
    gb_ref[...] = g[:, HALF:]
    dinv_ref[...] = dinv


_mm_call = pl.pallas_call(
    _mm_body,
    grid=(_GRID,),
    in_specs=[
        pl.BlockSpec((_ROWS, 1), lambda i: (i, 0)),
        pl.BlockSpec((_ROWS, 1), lambda i: (i, 0)),
        pl.BlockSpec((_ROWS, D_F), lambda i: (i, 0)),
        pl.BlockSpec((D_F, D_F), lambda i: (0, 0)),
    ],
    out_specs=[
        pl.BlockSpec((_ROWS, HALF), lambda i: (i, 0)),
        pl.BlockSpec((_ROWS, HALF), lambda i: (i, 0)),
        pl.BlockSpec((_ROWS, 1), lambda i: (i, 0)),
    ],
    out_shape=[
        jax.ShapeDtypeStruct((N_PAD, HALF), jnp.float32),
        jax.ShapeDtypeStruct((N_PAD, HALF), jnp.float32),
        jax.ShapeDtypeStruct((N_PAD, 1), jnp.float32),
    ],
)


def _post_body(x_ref, aa_ref, ab_ref, dinv_ref, bc_ref, lg_ref, lb_ref,
               wnn_ref, bnn_ref, out_ref):
    agg = jnp.concatenate([aa_ref[...], ab_ref[...]], axis=1)
    pre = agg * dinv_ref[...] + bc_ref[...]
    mu = jnp.mean(pre, axis=1, keepdims=True)
    d = pre - mu
    var = jnp.mean(d * d, axis=1, keepdims=True)
    ln = d * lax.rsqrt(var + 1e-5) * lg_ref[...] + lb_ref[...]
    r = jnp.maximum(x_ref[...] + ln, 0.0)
    out_ref[...] = (
        jnp.dot(r, wnn_ref[...], preferred_element_type=jnp.float32)
        + bnn_ref[...]
    )


_post_call = pl.pallas_call(
    _post_body,
    grid=(_GRID,),
    in_specs=[
        pl.BlockSpec((_ROWS, D_F), lambda i: (i, 0)),
        pl.BlockSpec((_ROWS, HALF), lambda i: (i, 0)),
        pl.BlockSpec((_ROWS, HALF), lambda i: (i, 0)),
        pl.BlockSpec((_ROWS, 1), lambda i: (i, 0)),
        pl.BlockSpec((1, D_F), lambda i: (0, 0)),
        pl.BlockSpec((1, D_F), lambda i: (0, 0)),
        pl.BlockSpec((1, D_F), lambda i: (0, 0)),
        pl.BlockSpec((D_F, D_F), lambda i: (0, 0)),
        pl.BlockSpec((1, D_F), lambda i: (0, 0)),
    ],
    out_specs=pl.BlockSpec((_ROWS, D_F), lambda i: (i, 0)),
    out_shape=jax.ShapeDtypeStruct((N_PAD, D_F), jnp.float32),
)


def kernel(x, edge_index, W_conv, b_conv, ln_gamma, ln_beta, W_nn, b_nn):
    ei = edge_index.astype(jnp.int32)
    src2d = jnp.concatenate(
        [ei[0], jnp.zeros((E_PAD - N_E,), jnp.int32)]).reshape(E_PAD // CH, CH)
    dst2d = jnp.concatenate(
        [ei[1], jnp.full((E_PAD - N_E,), N_PAD - 1, jnp.int32)]
    ).reshape(E_PAD // CH, CH)

    x_pad = jnp.pad(x, ((0, N_PAD - N_N), (0, 0)))
    deg0, deg1 = _deg_call(dst2d)
    ga, gb, dinv = _mm_call(
        deg0.reshape(N_PAD, 1), deg1.reshape(N_PAD, 1), x_pad, W_conv)
    agga, aggb = _agg_call(src2d, dst2d, ga, gb)
    out = _post_call(
        x_pad, agga, aggb, dinv,
        b_conv.reshape(1, D_F), ln_gamma.reshape(1, D_F),
        ln_beta.reshape(1, D_F), W_nn, b_nn.reshape(1, D_F),
    )
    return out[:N_N]


# speedup vs baseline: 1.1321x; 1.0111x over previous
import functools

import jax
import jax.numpy as jnp
from jax import lax
from jax.experimental import pallas as pl
from jax.experimental.pallas import tpu as pltpu
from jax.experimental.pallas import tpu_sc as plsc

N_N = 10000
D_F = 256
N_E = 160000
HALF = 128
N_PAD = 10240
E_PAD = 163840
CH = 128
CPT = E_PAD // CH // 16
RPT = N_PAD // 16
RCH = RPT // CH

_SC_MESH = plsc.VectorSubcoreMesh(core_axis_name="c", subcore_axis_name="s")


def _fill(ref, n, value):
    val = jnp.full((16,), value, jnp.float32)

    def body(i, _):
        ref[pl.ds(i * 16, 16)] = val
        return 0

    lax.fori_loop(0, n // 16, body, 0)


CPT_DEG = CPT // 2


def _deg_body(dst2d, deg0_out, deg1_out, didx, ones, dbuf, shared_deg):
    c = lax.axis_index("c")
    s = lax.axis_index("s")

    def run(deg_out):
        _fill(ones, CH, 1.0)
        _fill(dbuf, RPT, 0.0)
        pltpu.sync_copy(dbuf, shared_deg.at[pl.ds(s * RPT, RPT)])
        base = c * (E_PAD // CH // 2) + s * CPT_DEG
        pltpu.sync_copy(dst2d.at[pl.ds(base, CPT_DEG), :], didx)
        plsc.subcore_barrier()

        def body(j, _):
            pltpu.sync_copy(ones, shared_deg.at[didx.at[j]], add=True)
            return 0

        lax.fori_loop(0, CPT_DEG, body, 0)
        plsc.subcore_barrier()
        pltpu.sync_copy(shared_deg.at[pl.ds(s * RPT, RPT)], dbuf)
        pltpu.sync_copy(dbuf, deg_out.at[pl.ds(s * RPT, RPT)])

    @pl.when(c == 0)
    def _():
        run(deg0_out)

    @pl.when(c == 1)
    def _():
        run(deg1_out)


_deg_call = functools.partial(
    pl.kernel,
    out_type=(
        jax.ShapeDtypeStruct((N_PAD,), jnp.float32),
        jax.ShapeDtypeStruct((N_PAD,), jnp.float32),
    ),
    mesh=_SC_MESH,
    scratch_types=[
        pltpu.VMEM((CPT_DEG, CH), jnp.int32),
        pltpu.VMEM((CH,), jnp.float32),
        pltpu.VMEM((RPT,), jnp.float32),
        pltpu.VMEM_SHARED((N_PAD,), jnp.float32),
    ],
)(_deg_body)


PHC = CPT // 2


def _agg_body(src2d, dst2d, ga, gb, agga, aggb, sidx, didx, rows0, rows1,
              shared_agg, gsem0, gsem1):
    c = lax.axis_index("c")
    s = lax.axis_index("s")

    def run(g_hbm, agg_hbm):
        base = s * RPT
        pltpu.sync_copy(g_hbm.at[pl.ds(base, RPT), :],
                        shared_agg.at[pl.ds(base, RPT), :])
        plsc.subcore_barrier()

        def phase(p, _):
            base = s * CPT + p * PHC
            pltpu.sync_copy(src2d.at[pl.ds(base, PHC), :], sidx)
            pltpu.sync_copy(dst2d.at[pl.ds(base, PHC), :], didx)
            pltpu.async_copy(g_hbm.at[sidx.at[0]], rows0, gsem0)
            pltpu.async_copy(g_hbm.at[sidx.at[1]], rows1, gsem1)

            def body(i, _):
                j0 = 2 * i
                j1 = j0 + 1
                pltpu.make_async_copy(
                    g_hbm.at[sidx.at[j0]], rows0, gsem0).wait()
                pltpu.sync_copy(rows0, shared_agg.at[didx.at[j0]], add=True)

                @pl.when(j0 + 2 < PHC)
                def _():
                    pltpu.async_copy(g_hbm.at[sidx.at[j0 + 2]], rows0, gsem0)

                pltpu.make_async_copy(
                    g_hbm.at[sidx.at[j1]], rows1, gsem1).wait()
                pltpu.sync_copy(rows1, shared_agg.at[didx.at[j1]], add=True)

                @pl.when(j1 + 2 < PHC)
                def _():
                    pltpu.async_copy(g_hbm.at[sidx.at[j1 + 2]], rows1, gsem1)

                return 0

            lax.fori_loop(0, PHC // 2, body, 0)
            return 0

        lax.fori_loop(0, CPT // PHC, phase, 0)
        plsc.subcore_barrier()

        pltpu.sync_copy(shared_agg.at[pl.ds(base, RPT), :],
                        agg_hbm.at[pl.ds(base, RPT), :])

    @pl.when(c == 0)
    def _():
        run(ga, agga)

    @pl.when(c == 1)
    def _():
        run(gb, aggb)


_agg_call = functools.partial(
    pl.kernel,
    out_type=(
        jax.ShapeDtypeStruct((N_PAD, HALF), jnp.float32),
        jax.ShapeDtypeStruct((N_PAD, HALF), jnp.float32),
    ),
    mesh=_SC_MESH,
    scratch_types=[
        pltpu.VMEM((PHC, CH), jnp.int32),
        pltpu.VMEM((PHC, CH), jnp.int32),
        pltpu.VMEM((CH, HALF), jnp.float32),
        pltpu.VMEM((CH, HALF), jnp.float32),
        pltpu.VMEM_SHARED((N_PAD, HALF), jnp.float32),
        pltpu.SemaphoreType.DMA,
        pltpu.SemaphoreType.DMA,
    ],
)(_agg_body)


_ROWS = 2048
_GRID = N_PAD // _ROWS


def _mm_body(deg0_ref, deg1_ref, x_ref, w_ref, ga_ref, gb_ref, dinv_ref):
    dinv = lax.rsqrt(deg0_ref[...] + deg1_ref[...] + 1.0)
    h = jnp.dot(x_ref[...], w_ref[...], preferred_element_type=jnp.float32)
    g = h * dinv
    ga_ref[...] = g[:, :HALF]
    gb_ref[...] = g[:, HALF:]
    dinv_ref[...] = dinv


_mm_call = pl.pallas_call(
    _mm_body,
    grid=(_GRID,),
    in_specs=[
        pl.BlockSpec((_ROWS, 1), lambda i: (i, 0)),
        pl.BlockSpec((_ROWS, 1), lambda i: (i, 0)),
        pl.BlockSpec((_ROWS, D_F), lambda i: (i, 0)),
        pl.BlockSpec((D_F, D_F), lambda i: (0, 0)),
    ],
    out_specs=[
        pl.BlockSpec((_ROWS, HALF), lambda i: (i, 0)),
        pl.BlockSpec((_ROWS, HALF), lambda i: (i, 0)),
        pl.BlockSpec((_ROWS, 1), lambda i: (i, 0)),
    ],
    out_shape=[
        jax.ShapeDtypeStruct((N_PAD, HALF), jnp.float32),
        jax.ShapeDtypeStruct((N_PAD, HALF), jnp.float32),
        jax.ShapeDtypeStruct((N_PAD, 1), jnp.float32),
    ],
)


def _post_body(x_ref, aa_ref, ab_ref, dinv_ref, bc_ref, lg_ref, lb_ref,
               wnn_ref, bnn_ref, out_ref):
    agg = jnp.concatenate([aa_ref[...], ab_ref[...]], axis=1)
    pre = agg * dinv_ref[...] + bc_ref[...]
    mu = jnp.mean(pre, axis=1, keepdims=True)
    d = pre - mu
    var = jnp.mean(d * d, axis=1, keepdims=True)
    ln = d * lax.rsqrt(var + 1e-5) * lg_ref[...] + lb_ref[...]
    r = jnp.maximum(x_ref[...] + ln, 0.0)
    out_ref[...] = (
        jnp.dot(r, wnn_ref[...], preferred_element_type=jnp.float32)
        + bnn_ref[...]
    )


_post_call = pl.pallas_call(
    _post_body,
    grid=(_GRID,),
    in_specs=[
        pl.BlockSpec((_ROWS, D_F), lambda i: (i, 0)),
        pl.BlockSpec((_ROWS, HALF), lambda i: (i, 0)),
        pl.BlockSpec((_ROWS, HALF), lambda i: (i, 0)),
        pl.BlockSpec((_ROWS, 1), lambda i: (i, 0)),
        pl.BlockSpec((1, D_F), lambda i: (0, 0)),
        pl.BlockSpec((1, D_F), lambda i: (0, 0)),
        pl.BlockSpec((1, D_F), lambda i: (0, 0)),
        pl.BlockSpec((D_F, D_F), lambda i: (0, 0)),
        pl.BlockSpec((1, D_F), lambda i: (0, 0)),
    ],
    out_specs=pl.BlockSpec((_ROWS, D_F), lambda i: (i, 0)),
    out_shape=jax.ShapeDtypeStruct((N_PAD, D_F), jnp.float32),
)


def kernel(x, edge_index, W_conv, b_conv, ln_gamma, ln_beta, W_nn, b_nn):
    ei = edge_index.astype(jnp.int32)
    src2d = jnp.concatenate(
        [ei[0], jnp.zeros((E_PAD - N_E,), jnp.int32)]).reshape(E_PAD // CH, CH)
    dst2d = jnp.concatenate(
        [ei[1], jnp.full((E_PAD - N_E,), N_PAD - 1, jnp.int32)]
    ).reshape(E_PAD // CH, CH)

    x_pad = jnp.pad(x, ((0, N_PAD - N_N), (0, 0)))
    deg0, deg1 = _deg_call(dst2d)
    ga, gb, dinv = _mm_call(
        deg0.reshape(N_PAD, 1), deg1.reshape(N_PAD, 1), x_pad, W_conv)
    agga, aggb = _agg_call(src2d, dst2d, ga, gb)
    out = _post_call(
        x_pad, agga, aggb, dinv,
        b_conv.reshape(1, D_F), ln_gamma.reshape(1, D_F),
        ln_beta.reshape(1, D_F), W_nn, b_nn.reshape(1, D_F),
    )
    return out[:N_N]
